# baseline (speedup 1.0000x reference)
import functools

import jax
import jax.numpy as jnp
from jax import lax
from jax.experimental import pallas as pl
from jax.experimental.pallas import tpu as pltpu
from jax.experimental.pallas import tpu_sc as plsc

D = 64
NC = 2
NS = 16
NW = NC * NS
CHUNK = 128
K = 5
GROUP_ROWS = K * CHUNK


@functools.cache
def _build(total):
    per_w = total // NW
    nchunk = per_w // CHUNK
    ngroups = nchunk // K
    npair = ngroups // 2
    mesh = plsc.VectorSubcoreMesh(core_axis_name="c", subcore_axis_name="s")

    @functools.partial(
        pl.kernel,
        mesh=mesh,
        out_type=jax.ShapeDtypeStruct((NW, ngroups, GROUP_ROWS, 2, D),
                                      jnp.float32),
        scratch_types=[
            pltpu.VMEM((nchunk, CHUNK), jnp.int32),
            pltpu.VMEM((GROUP_ROWS, D), jnp.float32),
            pltpu.VMEM((GROUP_ROWS, D), jnp.float32),
            pltpu.SemaphoreType.DMA,
            pltpu.SemaphoreType.DMA,
            pltpu.SemaphoreType.DMA,
            pltpu.SemaphoreType.DMA,
        ],
        compiler_params=pltpu.CompilerParams(use_tc_tiling_on_sc=False),
    )
    def gather_kernel(idx_hbm, table_hbm, out_hbm, idx_v, buf0, buf1,
                      sg0, sg1, ss0, ss1):
        wid = lax.axis_index("s") * NC + lax.axis_index("c")
        pltpu.sync_copy(idx_hbm.at[wid], idx_v)
        bufs, sgs, sss = (buf0, buf1), (sg0, sg1), (ss0, ss1)

        def pair_body(p, carry):
            copies = []
            for s in range(2):
                g = 2 * p + s
                @pl.when(p > 0)
                def _():
                    pltpu.make_async_copy(
                        bufs[s], out_hbm.at[wid, g - 2, :, 0, :],
                        sss[s]).wait()
                copies.append([
                    pltpu.async_copy(
                        table_hbm.at[idx_v.at[g * K + b]],
                        bufs[s].at[pl.ds(b * CHUNK, CHUNK)], sgs[s])
                    for b in range(K)
                ])
            for s in range(2):
                g = 2 * p + s
                for c in copies[s]:
                    c.wait()
                pltpu.async_copy(bufs[s], out_hbm.at[wid, g, :, 0, :], sss[s])
            return carry

        lax.fori_loop(0, npair, pair_body, 0)
        for s in range(2):
            pltpu.make_async_copy(
                bufs[s], out_hbm.at[wid, ngroups - 2 + s, :, 0, :],
                sss[s]).wait()

    return gather_kernel


def kernel(inp, weight):
    total = inp.shape[0] * inp.shape[1]
    nchunk = total // NW // CHUNK
    idx = inp.astype(jnp.int32).reshape(NW, nchunk, CHUNK)
    out = _build(total)(idx, weight)
    out = out.reshape(total, 2, weight.shape[1])[:, 0, :]
    return out.reshape(inp.shape[0], inp.shape[1], weight.shape[1])


# speedup vs baseline: 2.5683x; 2.5683x over previous
import functools

import jax
import jax.numpy as jnp
from jax import lax
from jax.experimental import pallas as pl
from jax.experimental.pallas import tpu as pltpu
from jax.experimental.pallas import tpu_sc as plsc

D = 64
NC = 2
NS = 16
NW = NC * NS
CHUNK = 128
K = 2
NSETS = 4
GROUP_ROWS = K * CHUNK


@functools.cache
def _build(total):
    per_w = total // NW
    nchunk = per_w // CHUNK
    ngroups = nchunk // K
    nround = ngroups // NSETS
    mesh = plsc.VectorSubcoreMesh(core_axis_name="c", subcore_axis_name="s")

    @functools.partial(
        pl.kernel,
        mesh=mesh,
        out_type=jax.ShapeDtypeStruct((NW, ngroups, GROUP_ROWS, D), jnp.float32),
        scratch_types=[
            pltpu.VMEM((nchunk, CHUNK), jnp.int32),
        ] + [pltpu.VMEM((GROUP_ROWS, D), jnp.float32)] * NSETS
          + [pltpu.SemaphoreType.DMA] * (2 * NSETS),
        compiler_params=pltpu.CompilerParams(use_tc_tiling_on_sc=False),
    )
    def gather_kernel(idx_hbm, table_hbm, out_hbm, idx_v, *bufsem):
        bufs = bufsem[:NSETS]
        sgs = bufsem[NSETS:2 * NSETS]
        sss = bufsem[2 * NSETS:]
        wid = lax.axis_index("s") * NC + lax.axis_index("c")
        pltpu.sync_copy(idx_hbm.at[wid], idx_v)

        def round_body(p, carry):
            copies = []
            for s in range(NSETS):
                g = NSETS * p + s
                @pl.when(p > 0)
                def _():
                    pltpu.make_async_copy(
                        bufs[s], out_hbm.at[wid, g - NSETS], sss[s]).wait()
                copies.append([
                    pltpu.async_copy(
                        table_hbm.at[idx_v.at[g * K + b]],
                        bufs[s].at[pl.ds(b * CHUNK, CHUNK)], sgs[s])
                    for b in range(K)
                ])
            for s in range(NSETS):
                g = NSETS * p + s
                for c in copies[s]:
                    c.wait()
                pltpu.async_copy(bufs[s], out_hbm.at[wid, g], sss[s])
            return carry

        lax.fori_loop(0, nround, round_body, 0)
        for s in range(NSETS):
            pltpu.make_async_copy(
                bufs[s], out_hbm.at[wid, ngroups - NSETS + s], sss[s]).wait()

    return gather_kernel


def kernel(inp, weight):
    total = inp.shape[0] * inp.shape[1]
    nchunk = total // NW // CHUNK
    idx = inp.astype(jnp.int32).reshape(NW, nchunk, CHUNK)
    out = _build(total)(idx, weight)
    return out.reshape(inp.shape[0], inp.shape[1], weight.shape[1])
